# baseline (speedup 1.0000x reference)
import functools

import jax
import jax.numpy as jnp
from jax import lax
from jax.experimental import pallas as pl
from jax.experimental.pallas import tpu as pltpu
from jax.experimental.pallas import tpu_sc as plsc

_B, _L, _W, _V = 4, 2048, 2048, 2048
_ROWS = _B * _L
_NC, _NS = 2, 16
_NW = _NC * _NS
_RPW = _ROWS // _NW
_CHUNK = 16
_NCHUNK = _RPW // _CHUNK
_NBUF = 3


def _build():
    mesh = plsc.VectorSubcoreMesh(core_axis_name="c", subcore_axis_name="s")

    @functools.partial(
        pl.kernel,
        mesh=mesh,
        out_type=jax.ShapeDtypeStruct((_ROWS, _W), jnp.float32),
        scratch_types=[
            pltpu.VMEM((_RPW,), jnp.int32),
            pltpu.VMEM((_RPW,), jnp.int32),
            pltpu.VMEM((_NCHUNK, _CHUNK), jnp.int32),
            pltpu.VMEM((_NBUF, _CHUNK, _W), jnp.float32),
            pltpu.SemaphoreType.DMA,
            pltpu.SemaphoreType.DMA,
            pltpu.SemaphoreType.DMA,
            pltpu.SemaphoreType.DMA,
            pltpu.SemaphoreType.DMA,
            pltpu.SemaphoreType.DMA,
        ],
    )
    def k(table_hbm, idx_hbm, mask_hbm, out_hbm,
          idx_v, mask_v, eff_v, rows_v, g0, g1, g2, p0, p1, p2):
        gsem = (g0, g1, g2)
        psem = (p0, p1, p2)
        wid = lax.axis_index("s") * _NC + lax.axis_index("c")
        base = wid * _RPW

        pltpu.sync_copy(idx_hbm.at[pl.ds(base, _RPW)], idx_v)
        pltpu.sync_copy(mask_hbm.at[pl.ds(base, _RPW)], mask_v)

        for g in range(_NCHUNK):
            i = idx_v[pl.ds(g * _CHUNK, _CHUNK)]
            m = mask_v[pl.ds(g * _CHUNK, _CHUNK)]
            eff_v[g, pl.ds(0, _CHUNK)] = jnp.where(m != 0, i, _V)

        dg = [None] * _NCHUNK
        dp = [None] * _NCHUNK
        for c in range(_NBUF - 1):
            dg[c] = pltpu.async_copy(
                table_hbm.at[eff_v.at[c]], rows_v.at[c], gsem[c])
        for c in range(_NCHUNK):
            b = c % _NBUF
            dg[c].wait()
            dp[c] = pltpu.async_copy(
                rows_v.at[b], out_hbm.at[pl.ds(base + c * _CHUNK, _CHUNK)],
                psem[b])
            nxt = c + _NBUF - 1
            if nxt < _NCHUNK:
                b2 = nxt % _NBUF
                if c >= 1:
                    dp[c - 1].wait()
                dg[nxt] = pltpu.async_copy(
                    table_hbm.at[eff_v.at[nxt]], rows_v.at[b2], gsem[b2])
        for c in range(_NCHUNK - _NBUF, _NCHUNK):
            dp[c].wait()

    return k


_K = _build()


def kernel(obs_pos, obs_mask, table):
    idx = obs_pos.reshape(_ROWS).astype(jnp.int32)
    mask = obs_mask.reshape(_ROWS).astype(jnp.int32)
    table_z = jnp.concatenate(
        [table, jnp.zeros((1, _W), jnp.float32)], axis=0)
    out = _K(table_z, idx, mask)
    return out.reshape(_B, _L, _W)


# speedup vs baseline: 1.0313x; 1.0313x over previous
import functools

import jax
import jax.numpy as jnp
from jax import lax
from jax.experimental import pallas as pl
from jax.experimental.pallas import tpu as pltpu
from jax.experimental.pallas import tpu_sc as plsc

_B, _L, _W, _V = 4, 2048, 2048, 2048
_ROWS = _B * _L
_NC, _NS = 2, 16
_NW = _NC * _NS
_RPW = _ROWS // _NW
_CHUNK = 16
_NCHUNK = _RPW // _CHUNK
_NPAIR = _NCHUNK // 2


def _build():
    mesh = plsc.VectorSubcoreMesh(core_axis_name="c", subcore_axis_name="s")

    @functools.partial(
        pl.kernel,
        mesh=mesh,
        out_type=jax.ShapeDtypeStruct((_ROWS, _W), jnp.float32),
        scratch_types=[
            pltpu.VMEM((_RPW,), jnp.int32),
            pltpu.VMEM((_RPW,), jnp.int32),
            pltpu.VMEM((_CHUNK, _W), jnp.float32),
            pltpu.VMEM((_CHUNK, _W), jnp.float32),
            pltpu.SemaphoreType.DMA,
            pltpu.SemaphoreType.DMA,
            pltpu.SemaphoreType.DMA,
            pltpu.SemaphoreType.DMA,
        ],
    )
    def k(table_hbm, idx_hbm, mask_hbm, out_hbm,
          idx_v, mask_v, buf_a, buf_b, ga, gb, pa, pb):
        wid = lax.axis_index("s") * _NC + lax.axis_index("c")
        base = wid * _RPW

        pltpu.sync_copy(idx_hbm.at[pl.ds(base, _RPW)], idx_v)
        pltpu.sync_copy(mask_hbm.at[pl.ds(base, _RPW)], mask_v)

        def fire_chunk(c, buf, sem):
            i = idx_v[pl.ds(c * _CHUNK, _CHUNK)]
            m = mask_v[pl.ds(c * _CHUNK, _CHUNK)]
            eff = jnp.where(m != 0, i, _V)
            for l in range(_CHUNK):
                pltpu.async_copy(
                    table_hbm.at[pl.ds(eff[l], 1)],
                    buf.at[pl.ds(l, 1)], sem)

        def drain_gather(buf, sem):
            pltpu.make_async_copy(
                table_hbm.at[pl.ds(0, _CHUNK)], buf, sem).wait()

        def fire_wb(c, buf, sem):
            pltpu.async_copy(
                buf, out_hbm.at[pl.ds(base + c * _CHUNK, _CHUNK)], sem)

        def drain_wb(buf, sem):
            pltpu.make_async_copy(
                buf, out_hbm.at[pl.ds(0, _CHUNK)], sem).wait()

        fire_chunk(0, buf_a, ga)

        @pl.loop(0, _NPAIR)
        def _pair(g):
            c0 = 2 * g
            c1 = c0 + 1
            drain_gather(buf_a, ga)
            fire_chunk(c1, buf_b, gb)
            fire_wb(c0, buf_a, pa)
            drain_gather(buf_b, gb)
            drain_wb(buf_a, pa)
            fire_wb(c1, buf_b, pb)

            @pl.when(g < _NPAIR - 1)
            def _():
                fire_chunk(c0 + 2, buf_a, ga)

            drain_wb(buf_b, pb)

    return k


_K = _build()


def kernel(obs_pos, obs_mask, table):
    idx = obs_pos.reshape(_ROWS).astype(jnp.int32)
    mask = obs_mask.reshape(_ROWS).astype(jnp.int32)
    table_z = jnp.concatenate(
        [table, jnp.zeros((1, _W), jnp.float32)], axis=0)
    out = _K(table_z, idx, mask)
    return out.reshape(_B, _L, _W)


# speedup vs baseline: 1.1676x; 1.1321x over previous
import functools

import jax
import jax.numpy as jnp
from jax import lax
from jax.experimental import pallas as pl
from jax.experimental.pallas import tpu as pltpu
from jax.experimental.pallas import tpu_sc as plsc

_B, _L, _W, _V = 4, 2048, 2048, 2048
_ROWS = _B * _L
_NC, _NS = 2, 16
_NW = _NC * _NS
_RPW = _ROWS // _NW
_CHUNK = 16
_NCHUNK = _RPW // _CHUNK
_NPAIR = _NCHUNK // 2


def _build():
    mesh = plsc.VectorSubcoreMesh(core_axis_name="c", subcore_axis_name="s")

    @functools.partial(
        pl.kernel,
        mesh=mesh,
        out_type=jax.ShapeDtypeStruct((_ROWS, _W), jnp.float32),
        scratch_types=[
            pltpu.VMEM((_RPW,), jnp.int32),
            pltpu.VMEM((_RPW,), jnp.int32),
            pltpu.VMEM_SHARED((_NS, _CHUNK, _W), jnp.float32),
            pltpu.VMEM_SHARED((_NS, _CHUNK, _W), jnp.float32),
            pltpu.SemaphoreType.DMA,
            pltpu.SemaphoreType.DMA,
            pltpu.SemaphoreType.DMA,
            pltpu.SemaphoreType.DMA,
        ],
    )
    def k(table_hbm, idx_hbm, mask_hbm, out_hbm,
          idx_v, mask_v, sh_a, sh_b, ga, gb, pa, pb):
        sid = lax.axis_index("s")
        wid = sid * _NC + lax.axis_index("c")
        base = wid * _RPW
        buf_a = sh_a.at[sid]
        buf_b = sh_b.at[sid]

        pltpu.sync_copy(idx_hbm.at[pl.ds(base, _RPW)], idx_v)
        pltpu.sync_copy(mask_hbm.at[pl.ds(base, _RPW)], mask_v)

        def fire_chunk(c, buf, sem):
            i = idx_v[pl.ds(c * _CHUNK, _CHUNK)]
            m = mask_v[pl.ds(c * _CHUNK, _CHUNK)]
            eff = jnp.where(m != 0, i, _V)
            for l in range(_CHUNK):
                pltpu.async_copy(
                    table_hbm.at[pl.ds(eff[l], 1)],
                    buf.at[pl.ds(l, 1)], sem)

        def drain_gather(buf, sem):
            pltpu.make_async_copy(
                table_hbm.at[pl.ds(0, _CHUNK)], buf, sem).wait()

        def fire_wb(c, buf, sem):
            pltpu.async_copy(
                buf, out_hbm.at[pl.ds(base + c * _CHUNK, _CHUNK)], sem)

        def drain_wb(buf, sem):
            pltpu.make_async_copy(
                buf, out_hbm.at[pl.ds(0, _CHUNK)], sem).wait()

        fire_chunk(0, buf_a, ga)

        @pl.loop(0, _NPAIR)
        def _pair(g):
            c0 = 2 * g
            c1 = c0 + 1
            drain_gather(buf_a, ga)
            fire_chunk(c1, buf_b, gb)
            fire_wb(c0, buf_a, pa)
            drain_gather(buf_b, gb)
            drain_wb(buf_a, pa)
            fire_wb(c1, buf_b, pb)

            @pl.when(g < _NPAIR - 1)
            def _():
                fire_chunk(c0 + 2, buf_a, ga)

            drain_wb(buf_b, pb)

    return k


_K = _build()


def kernel(obs_pos, obs_mask, table):
    idx = obs_pos.reshape(_ROWS).astype(jnp.int32)
    mask = obs_mask.reshape(_ROWS).astype(jnp.int32)
    table_z = jnp.concatenate(
        [table, jnp.zeros((1, _W), jnp.float32)], axis=0)
    out = _K(table_z, idx, mask)
    return out.reshape(_B, _L, _W)


# speedup vs baseline: 5.1979x; 4.4520x over previous
import functools

import jax
import jax.numpy as jnp
from jax import lax
from jax.experimental import pallas as pl
from jax.experimental.pallas import tpu as pltpu
from jax.experimental.pallas import tpu_sc as plsc

_B, _L, _W, _V = 4, 2048, 2048, 2048
_ROWS = _B * _L
_NC, _NS = 2, 16
_NW = _NC * _NS
_RPW = _ROWS // _NW
_CHUNK = 16
_NCHUNK = _RPW // _CHUNK
_NPAIR = _NCHUNK // 2


def _build():
    mesh = plsc.VectorSubcoreMesh(core_axis_name="c", subcore_axis_name="s")

    @functools.partial(
        pl.kernel,
        mesh=mesh,
        out_type=jax.ShapeDtypeStruct((_ROWS, _W), jnp.float32),
        scratch_types=[
            pltpu.VMEM((_RPW,), jnp.int32),
            pltpu.VMEM((_RPW,), jnp.int32),
            pltpu.VMEM((1, _W), jnp.float32),
            pltpu.VMEM_SHARED((_NS, _CHUNK, _W), jnp.float32),
            pltpu.VMEM_SHARED((_NS, _CHUNK, _W), jnp.float32),
            pltpu.SemaphoreType.DMA,
            pltpu.SemaphoreType.DMA,
            pltpu.SemaphoreType.DMA,
            pltpu.SemaphoreType.DMA,
            pltpu.SemaphoreType.DMA,
        ],
    )
    def k(table_hbm, idx_hbm, mask_hbm, out_hbm,
          idx_v, mask_v, zrow, sh_a, sh_b, ga, gb, pa, pb, zs):
        sid = lax.axis_index("s")
        wid = sid * _NC + lax.axis_index("c")
        base = wid * _RPW
        buf_a = sh_a.at[sid]
        buf_b = sh_b.at[sid]

        pltpu.sync_copy(idx_hbm.at[pl.ds(base, _RPW)], idx_v)
        pltpu.sync_copy(mask_hbm.at[pl.ds(base, _RPW)], mask_v)

        zv = jnp.zeros((16,), jnp.float32)
        for j in range(_W // 16):
            zrow[0, pl.ds(j * 16, 16)] = zv

        def vecs(c):
            i = idx_v[pl.ds(c * _CHUNK, _CHUNK)]
            m = mask_v[pl.ds(c * _CHUNK, _CHUNK)]
            return i, m

        def fire_chunk(c, buf, sem):
            i, m = vecs(c)
            for l in range(_CHUNK):

                @pl.when(m[l] != 0)
                def _():
                    pltpu.async_copy(
                        table_hbm.at[pl.ds(i[l], 1)],
                        buf.at[pl.ds(l, 1)], sem)

                @pl.when(m[l] == 0)
                def _():
                    pltpu.async_copy(
                        zrow,
                        out_hbm.at[pl.ds(base + c * _CHUNK + l, 1)], zs)

        def drain_gathers(c, buf, sem):
            _, m = vecs(c)
            for l in range(_CHUNK):

                @pl.when(m[l] != 0)
                def _():
                    pltpu.make_async_copy(
                        table_hbm.at[pl.ds(0, 1)],
                        buf.at[pl.ds(l, 1)], sem).wait()

        def fire_wb(c, buf, sem):
            _, m = vecs(c)
            for l in range(_CHUNK):

                @pl.when(m[l] != 0)
                def _():
                    pltpu.async_copy(
                        buf.at[pl.ds(l, 1)],
                        out_hbm.at[pl.ds(base + c * _CHUNK + l, 1)], sem)

        def drain_wb(c, buf, sem):
            _, m = vecs(c)
            for l in range(_CHUNK):

                @pl.when(m[l] != 0)
                def _():
                    pltpu.make_async_copy(
                        buf.at[pl.ds(l, 1)],
                        out_hbm.at[pl.ds(0, 1)], sem).wait()

        def drain_zeros(c):
            _, m = vecs(c)
            for l in range(_CHUNK):

                @pl.when(m[l] == 0)
                def _():
                    pltpu.make_async_copy(
                        zrow, out_hbm.at[pl.ds(0, 1)], zs).wait()

        fire_chunk(0, buf_a, ga)

        @pl.loop(0, _NPAIR)
        def _pair(g):
            c0 = 2 * g
            c1 = c0 + 1
            drain_gathers(c0, buf_a, ga)
            fire_chunk(c1, buf_b, gb)
            fire_wb(c0, buf_a, pa)
            drain_gathers(c1, buf_b, gb)
            drain_wb(c0, buf_a, pa)
            fire_wb(c1, buf_b, pb)

            @pl.when(g < _NPAIR - 1)
            def _():
                fire_chunk(c0 + 2, buf_a, ga)

            drain_wb(c1, buf_b, pb)
            drain_zeros(c0)
            drain_zeros(c1)

    return k


_K = _build()


def kernel(obs_pos, obs_mask, table):
    idx = obs_pos.reshape(_ROWS).astype(jnp.int32)
    mask = obs_mask.reshape(_ROWS).astype(jnp.int32)
    out = _K(table, idx, mask)
    return out.reshape(_B, _L, _W)
